# baseline (speedup 1.0000x reference)
import functools

import jax
import jax.numpy as jnp
from jax import lax
from jax.experimental import pallas as pl
from jax.experimental.pallas import tpu as pltpu
from jax.experimental.pallas import tpu_sc as plsc

_N = 100000
_E = 1600000
_PE = 16
_H = 64

_W = 128
_WB = 5
_EP = 1638400
_NBLK = _EP // (_W * _WB)
_NWORK = 32
_BPT = _NBLK // _NWORK
_CH = _WB * _W

_NS = 100096
_NC = 100352
_SST = _NS // 16
_CST = _NC // 16
_TQ = 368

_R = 2048


def _sc_segsum(glpe, row3d, col3d, zs, zc, ones):
    mesh = plsc.VectorSubcoreMesh(core_axis_name="c", subcore_axis_name="s")

    @functools.partial(
        pl.kernel,
        out_type=[
            jax.ShapeDtypeStruct((2, _PE, _NS), jnp.float32),
            jax.ShapeDtypeStruct((2, 16, _CST), jnp.float32),
        ],
        mesh=mesh,
        compiler_params=pltpu.CompilerParams(use_tc_tiling_on_sc=False,
                                             needs_layout_passes=False),
        scratch_types=[
            pltpu.VMEM((1, _WB, _W), jnp.int32),
            pltpu.VMEM((1, _WB, _W), jnp.int32),
            pltpu.VMEM((_CH, _PE), jnp.float32),
            pltpu.VMEM((_W,), jnp.float32),
            pltpu.VMEM((2, _PE, _TQ), jnp.float32),
            pltpu.VMEM_SHARED((_NS, _PE), jnp.float32),
            pltpu.VMEM_SHARED((_NC,), jnp.float32),
            pltpu.SemaphoreType.DMA,
            pltpu.SemaphoreType.DMA,
            pltpu.SemaphoreType.DMA,
        ],
    )
    def k(glpe_hbm, row_hbm, col_hbm, zs_hbm, zc_hbm, ones_hbm,
          s_out, c_out, row_v, col_v, rows_v, ones_v, t_v, s_sh, c_sh,
          gsem, ssem, csem):
        c = lax.axis_index("c")
        s = lax.axis_index("s")
        w = c * 16 + s
        pltpu.sync_copy(zs_hbm, s_sh.at[pl.ds(s * _SST, _SST)])
        pltpu.sync_copy(zc_hbm, c_sh.at[pl.ds(s * _CST, _CST)])
        pltpu.sync_copy(ones_hbm, ones_v)
        plsc.subcore_barrier()

        @pl.loop(0, _BPT)
        def _(sc):
            base = w * _BPT + sc
            pltpu.sync_copy(row_hbm.at[pl.ds(base, 1)], row_v)
            pltpu.sync_copy(col_hbm.at[pl.ds(base, 1)], col_v)

            @pl.loop(0, _WB)
            def _(j):
                pltpu.async_copy(glpe_hbm.at[row_v.at[0, j]],
                                 rows_v.at[pl.ds(j * _W, _W)], gsem)

            pltpu.make_async_copy(glpe_hbm.at[pl.ds(0, _CH)],
                                  rows_v, gsem).wait()

            @pl.loop(0, _WB)
            def _(j):
                src = rows_v.at[pl.ds(j * _W, _W)]
                pltpu.async_copy(src, s_sh.at[col_v.at[0, j]],
                                 ssem, add=True)
                pltpu.async_copy(ones_v, c_sh.at[col_v.at[0, j]],
                                 csem, add=True)

            pltpu.make_async_copy(rows_v, s_sh.at[pl.ds(0, _CH)],
                                  ssem).wait()
            pltpu.make_async_copy(zc_hbm.at[pl.ds(0, _CH)],
                                  c_sh.at[pl.ds(0, _CH)], csem).wait()

        plsc.subcore_barrier()
        lanes = lax.iota(jnp.int32, 16)

        def drain_chunk(j):
            roff = s * _SST + j * _TQ
            for k in range(16):
                pltpu.make_async_copy(
                    t_v.at[j % 2, k],
                    s_out.at[c, k, pl.ds(roff, _TQ)], gsem).wait()

        @pl.loop(0, 17)
        def _(q):
            @pl.when(q >= 2)
            def _():
                drain_chunk(q - 2)

            roff = s * _SST + q * _TQ
            pltpu.sync_copy(s_sh.at[pl.ds(roff, _TQ)],
                            rows_v.at[pl.ds(0, _TQ)])

            @pl.loop(0, _TQ)
            def _(r):
                plsc.store_scatter(t_v.at[q % 2],
                                   [lanes, jnp.full((16,), r, jnp.int32)],
                                   rows_v[r])

            for k in range(16):
                pltpu.async_copy(t_v.at[q % 2, k],
                                 s_out.at[c, k, pl.ds(roff, _TQ)], gsem)

        drain_chunk(15)
        drain_chunk(16)
        pltpu.sync_copy(c_sh.at[pl.ds(s * _CST, _CST)], c_out.at[c, s])

    return k(glpe, row3d, col3d, zs, zc, ones)


def _tc_combine(x_clique, deg2d, tree_lpe, seg, cnt, demb_pad,
                w1, b1, w2, b2, tw, tb, lw_aug):
    def body(x_ref, d_ref, t_ref, s_ref, c_ref, de_ref, w1_ref, b1_ref,
             w2_ref, b2_ref, tw_ref, tb_ref, lwa_ref, o_ref):
        t3 = jax.nn.relu(
            jnp.dot(de_ref[...], w1_ref[...],
                    preferred_element_type=jnp.float32) + b1_ref[...])
        t4 = jnp.dot(t3, w2_ref[...], preferred_element_type=jnp.float32)
        iota = lax.broadcasted_iota(jnp.int32, (_R, 128), 1)
        onehot = (d_ref[...] == iota).astype(jnp.float32)
        base = (jnp.dot(x_ref[...], w2_ref[...],
                        preferred_element_type=jnp.float32)
                + jnp.dot(onehot, t4, preferred_element_type=jnp.float32)
                + b2_ref[...])
        t = t_ref[...]
        t = jnp.where(t == t, t, 0.0)
        tree_pe = jnp.dot(t, tw_ref[...],
                          preferred_element_type=jnp.float32) + tb_ref[...]
        cn = c_ref[0] + c_ref[1]
        recip = 1.0 / jnp.maximum(cn, 1.0)
        mn = jnp.minimum(cn, 1.0)
        ssum = s_ref[0] + s_ref[1]
        aug = jnp.concatenate([ssum * recip, mn], axis=0)
        pe = lax.dot_general(aug, lwa_ref[...],
                             (((0,), (0,)), ((), ())),
                             preferred_element_type=jnp.float32)
        o_ref[...] = base + jnp.concatenate([pe, tree_pe], axis=1)

    full = lambda shape: pl.BlockSpec(shape, lambda i: (0,) * len(shape))
    return pl.pallas_call(
        body,
        grid=(49,),
        in_specs=[
            pl.BlockSpec((_R, _H), lambda i: (i, 0)),
            pl.BlockSpec((_R, 1), lambda i: (i, 0)),
            pl.BlockSpec((_R, _PE), lambda i: (i, 0)),
            pl.BlockSpec((2, _PE, _R), lambda i: (0, 0, i)),
            pl.BlockSpec((2, 1, _R), lambda i: (0, 0, i)),
            full((128, _H)),
            full((_H, _H)),
            full((1, _H)),
            full((_H, _H)),
            full((1, _H)),
            full((_PE, _H // 2)),
            full((1, _H // 2)),
            full((_PE + 1, _H // 2)),
        ],
        out_specs=pl.BlockSpec((_R, _H), lambda i: (i, 0)),
        out_shape=jax.ShapeDtypeStruct((_N, _H), jnp.float32),
    )(x_clique, deg2d, tree_lpe, seg, cnt, demb_pad,
      w1, b1, w2, b2, tw, tb, lw_aug)


def kernel(x_clique, tree_degree, tree_lpe, graph_lpe, atom2clique_index,
           deg_emb, deg_lin_w, deg_lin_b, deg_merge_w, deg_merge_b,
           tree_lpe_w, tree_lpe_b, lpe_w, lpe_b):
    pad = jnp.tile(jnp.array([[0], [_NS - 1]], jnp.int32), (1, _EP - _E))
    a2cp = jnp.concatenate([atom2clique_index, pad], axis=1)
    row3d = a2cp[0].reshape(_NBLK, _WB, _W)
    col3d = a2cp[1].reshape(_NBLK, _WB, _W)
    zs = jnp.zeros((_SST, _PE), jnp.float32)
    zc = jnp.zeros((_CST,), jnp.float32)
    ones = jnp.ones((_W,), jnp.float32)
    seg, cnt = _sc_segsum(graph_lpe, row3d, col3d, zs, zc, ones)
    cnt = cnt.reshape(2, 1, _NC)

    deg2d = tree_degree.reshape(_N, 1)
    demb_pad = jnp.zeros((128, _H), jnp.float32).at[:100].set(deg_emb)
    lw_aug = jnp.concatenate([lpe_w, lpe_b.reshape(1, _H // 2)], axis=0)
    return _tc_combine(
        x_clique, deg2d, tree_lpe, seg, cnt, demb_pad,
        deg_lin_w, deg_lin_b.reshape(1, _H),
        deg_merge_w, deg_merge_b.reshape(1, _H),
        tree_lpe_w, tree_lpe_b.reshape(1, _H // 2), lw_aug)


# speedup vs baseline: 1.0587x; 1.0587x over previous
import functools

import jax
import jax.numpy as jnp
from jax import lax
from jax.experimental import pallas as pl
from jax.experimental.pallas import tpu as pltpu
from jax.experimental.pallas import tpu_sc as plsc

_N = 100000
_E = 1600000
_PE = 16
_H = 64

_W = 128
_WB = 5
_EP = 1638400
_NBLK = _EP // (_W * _WB)
_NWORK = 32
_BPT = _NBLK // _NWORK
_CH = _WB * _W

_NS = 100096
_NC = 100352
_SST = _NS // 16
_CST = _NC // 16
_TQ = 368

_R = 2048


def _sc_segsum(glpe, row3d, col3d, zs, zc, ones):
    mesh = plsc.VectorSubcoreMesh(core_axis_name="c", subcore_axis_name="s")

    @functools.partial(
        pl.kernel,
        out_type=[
            jax.ShapeDtypeStruct((2, _PE, _NS), jnp.float32),
            jax.ShapeDtypeStruct((2, 16, _CST), jnp.float32),
        ],
        mesh=mesh,
        compiler_params=pltpu.CompilerParams(use_tc_tiling_on_sc=False,
                                             needs_layout_passes=False),
        scratch_types=[
            pltpu.VMEM((1, _WB, _W), jnp.int32),
            pltpu.VMEM((1, _WB, _W), jnp.int32),
            pltpu.VMEM((_CH, _PE), jnp.float32),
            pltpu.VMEM((_W,), jnp.float32),
            pltpu.VMEM((2, _PE, _TQ), jnp.float32),
            pltpu.VMEM_SHARED((_NS, _PE), jnp.float32),
            pltpu.VMEM_SHARED((_NC,), jnp.float32),
            pltpu.SemaphoreType.DMA,
            pltpu.SemaphoreType.DMA,
            pltpu.SemaphoreType.DMA,
        ],
    )
    def k(glpe_hbm, row_hbm, col_hbm, zs_hbm, zc_hbm, ones_hbm,
          s_out, c_out, row_v, col_v, rows_v, ones_v, t_v, s_sh, c_sh,
          gsem, ssem, csem):
        c = lax.axis_index("c")
        s = lax.axis_index("s")
        w = c * 16 + s
        pltpu.sync_copy(zs_hbm, s_sh.at[pl.ds(s * _SST, _SST)])
        pltpu.sync_copy(zc_hbm, c_sh.at[pl.ds(s * _CST, _CST)])
        pltpu.sync_copy(ones_hbm, ones_v)
        plsc.subcore_barrier()

        @pl.loop(0, _BPT)
        def _(sc):
            base = w * _BPT + sc
            pltpu.sync_copy(row_hbm.at[pl.ds(base, 1)], row_v)
            pltpu.sync_copy(col_hbm.at[pl.ds(base, 1)], col_v)

            @pl.loop(0, _WB)
            def _(j):
                pltpu.async_copy(glpe_hbm.at[row_v.at[0, j]],
                                 rows_v.at[pl.ds(j * _W, _W)], gsem)

            pltpu.make_async_copy(glpe_hbm.at[pl.ds(0, _CH)],
                                  rows_v, gsem).wait()

            @pl.loop(0, _WB)
            def _(j):
                src = rows_v.at[pl.ds(j * _W, _W)]
                pltpu.async_copy(src, s_sh.at[col_v.at[0, j]],
                                 ssem, add=True)
                pltpu.async_copy(ones_v, c_sh.at[col_v.at[0, j]],
                                 csem, add=True)

            pltpu.make_async_copy(rows_v, s_sh.at[pl.ds(0, _CH)],
                                  ssem).wait()
            pltpu.make_async_copy(zc_hbm.at[pl.ds(0, _CH)],
                                  c_sh.at[pl.ds(0, _CH)], csem).wait()

        plsc.subcore_barrier()
        lanes = lax.iota(jnp.int32, 16)

        def drain_chunk(j):
            roff = s * _SST + j * _TQ
            for k in range(16):
                pltpu.make_async_copy(
                    t_v.at[j % 2, k],
                    s_out.at[c, k, pl.ds(roff, _TQ)], gsem).wait()

        @pl.loop(0, 17)
        def _(q):
            @pl.when(q >= 2)
            def _():
                drain_chunk(q - 2)

            roff = s * _SST + q * _TQ
            pltpu.sync_copy(s_sh.at[pl.ds(roff, _TQ)],
                            rows_v.at[pl.ds(0, _TQ)])

            @pl.loop(0, _TQ)
            def _(r):
                plsc.store_scatter(t_v.at[q % 2],
                                   [lanes, jnp.full((16,), r, jnp.int32)],
                                   rows_v[r])

            for k in range(16):
                pltpu.async_copy(t_v.at[q % 2, k],
                                 s_out.at[c, k, pl.ds(roff, _TQ)], gsem)

        drain_chunk(15)
        drain_chunk(16)
        pltpu.sync_copy(c_sh.at[pl.ds(s * _CST, _CST)], c_out.at[c, s])

    return k(glpe, row3d, col3d, zs, zc, ones)


def _tc_combine(x_clique, deg2d, tree_lpe, seg, cnt, demb_pad,
                w1, b1, w2, b2, tw, tb, lw_aug):
    def body(x_ref, d_ref, t_ref, s_ref, c_ref, de_ref, w1_ref, b1_ref,
             w2_ref, b2_ref, tw_ref, tb_ref, lwa_ref, o_ref):
        t3 = jax.nn.relu(
            jnp.dot(de_ref[...], w1_ref[...],
                    preferred_element_type=jnp.float32) + b1_ref[...])
        t4 = jnp.dot(t3, w2_ref[...], preferred_element_type=jnp.float32)
        iota = lax.broadcasted_iota(jnp.int32, (128, _R), 0)
        onehot_t = (d_ref[...] == iota).astype(jnp.float32)
        deg_part = lax.dot_general(onehot_t, t4, (((0,), (0,)), ((), ())),
                                   preferred_element_type=jnp.float32)
        base = (jnp.dot(x_ref[...], w2_ref[...],
                        preferred_element_type=jnp.float32)
                + deg_part + b2_ref[...])
        t = t_ref[...]
        t = jnp.where(t == t, t, 0.0)
        tree_pe = jnp.dot(t, tw_ref[...],
                          preferred_element_type=jnp.float32) + tb_ref[...]
        cn = c_ref[0] + c_ref[1]
        recip = 1.0 / jnp.maximum(cn, 1.0)
        mn = jnp.minimum(cn, 1.0)
        ssum = s_ref[0] + s_ref[1]
        aug = jnp.concatenate([ssum * recip, mn], axis=0)
        pe = lax.dot_general(aug, lwa_ref[...],
                             (((0,), (0,)), ((), ())),
                             preferred_element_type=jnp.float32)
        o_ref[...] = base + jnp.concatenate([pe, tree_pe], axis=1)

    full = lambda shape: pl.BlockSpec(shape, lambda i: (0,) * len(shape))
    return pl.pallas_call(
        body,
        grid=(49,),
        in_specs=[
            pl.BlockSpec((_R, _H), lambda i: (i, 0)),
            pl.BlockSpec((1, _R), lambda i: (0, i)),
            pl.BlockSpec((_R, _PE), lambda i: (i, 0)),
            pl.BlockSpec((2, _PE, _R), lambda i: (0, 0, i)),
            pl.BlockSpec((2, 1, _R), lambda i: (0, 0, i)),
            full((128, _H)),
            full((_H, _H)),
            full((1, _H)),
            full((_H, _H)),
            full((1, _H)),
            full((_PE, _H // 2)),
            full((1, _H // 2)),
            full((_PE + 1, _H // 2)),
        ],
        out_specs=pl.BlockSpec((_R, _H), lambda i: (i, 0)),
        out_shape=jax.ShapeDtypeStruct((_N, _H), jnp.float32),
    )(x_clique, deg2d, tree_lpe, seg, cnt, demb_pad,
      w1, b1, w2, b2, tw, tb, lw_aug)


def kernel(x_clique, tree_degree, tree_lpe, graph_lpe, atom2clique_index,
           deg_emb, deg_lin_w, deg_lin_b, deg_merge_w, deg_merge_b,
           tree_lpe_w, tree_lpe_b, lpe_w, lpe_b):
    padc = _N + jnp.arange(_EP - _E, dtype=jnp.int32) % (_NS - _N)
    pad = jnp.stack([jnp.zeros(_EP - _E, jnp.int32), padc])
    a2cp = jnp.concatenate([atom2clique_index, pad], axis=1)
    row3d = a2cp[0].reshape(_NBLK, _WB, _W)
    col3d = a2cp[1].reshape(_NBLK, _WB, _W)
    zs = jnp.zeros((_SST, _PE), jnp.float32)
    zc = jnp.zeros((_CST,), jnp.float32)
    ones = jnp.ones((_W,), jnp.float32)
    seg, cnt = _sc_segsum(graph_lpe, row3d, col3d, zs, zc, ones)
    cnt = cnt.reshape(2, 1, _NC)

    deg2d = tree_degree.reshape(1, _N)
    demb_pad = jnp.zeros((128, _H), jnp.float32).at[:100].set(deg_emb)
    lw_aug = jnp.concatenate([lpe_w, lpe_b.reshape(1, _H // 2)], axis=0)
    return _tc_combine(
        x_clique, deg2d, tree_lpe, seg, cnt, demb_pad,
        deg_lin_w, deg_lin_b.reshape(1, _H),
        deg_merge_w, deg_merge_b.reshape(1, _H),
        tree_lpe_w, tree_lpe_b.reshape(1, _H // 2), lw_aug)


# speedup vs baseline: 1.4745x; 1.3928x over previous
import functools

import jax
import jax.numpy as jnp
from jax import lax
from jax.experimental import pallas as pl
from jax.experimental.pallas import tpu as pltpu
from jax.experimental.pallas import tpu_sc as plsc

_N = 100000
_E = 1600000
_PE = 16
_H = 64

_W = 128
_WB = 5
_EP = 1638400
_NBLK = _EP // (_W * _WB)
_NWORK = 32
_BPT = _NBLK // _NWORK
_CH = _WB * _W

_NS = 100096
_NC = 100352
_SST = _NS // 16
_CST = _NC // 16
_TQ = 368

_R = 2048


def _sc_segsum(glpe, row3d, col3d, zs, zc, ones):
    mesh = plsc.VectorSubcoreMesh(core_axis_name="c", subcore_axis_name="s")

    @functools.partial(
        pl.kernel,
        out_type=[
            jax.ShapeDtypeStruct((2, _PE, _NS), jnp.float32),
            jax.ShapeDtypeStruct((2, 16, _CST), jnp.float32),
        ],
        mesh=mesh,
        compiler_params=pltpu.CompilerParams(use_tc_tiling_on_sc=False,
                                             needs_layout_passes=False),
        scratch_types=[
            pltpu.VMEM((1, _WB, _W), jnp.int32),
            pltpu.VMEM((1, _WB, _W), jnp.int32),
            pltpu.VMEM((_CH, _PE), jnp.float32),
            pltpu.VMEM((_W,), jnp.float32),
            pltpu.VMEM((2, _PE, _TQ), jnp.float32),
            pltpu.VMEM_SHARED((_NS, _PE), jnp.float32),
            pltpu.VMEM_SHARED((_NC,), jnp.float32),
            pltpu.SemaphoreType.DMA,
            pltpu.SemaphoreType.DMA,
            pltpu.SemaphoreType.DMA,
        ],
    )
    def k(glpe_hbm, row_hbm, col_hbm, zs_hbm, zc_hbm, ones_hbm,
          s_out, c_out, row_v, col_v, rows_v, ones_v, t_v, s_sh, c_sh,
          gsem, ssem, csem):
        c = lax.axis_index("c")
        s = lax.axis_index("s")
        w = c * 16 + s
        pltpu.sync_copy(zs_hbm, s_sh.at[pl.ds(s * _SST, _SST)])
        pltpu.sync_copy(zc_hbm, c_sh.at[pl.ds(s * _CST, _CST)])
        pltpu.sync_copy(ones_hbm, ones_v)
        plsc.subcore_barrier()

        @pl.loop(0, _BPT)
        def _(sc):
            base = w * _BPT + sc
            pltpu.sync_copy(row_hbm.at[pl.ds(base, 1)], row_v)
            pltpu.sync_copy(col_hbm.at[pl.ds(base, 1)], col_v)

            @pl.loop(0, _WB)
            def _(j):
                pltpu.async_copy(glpe_hbm.at[row_v.at[0, j]],
                                 rows_v.at[pl.ds(j * _W, _W)], gsem)

            pltpu.make_async_copy(glpe_hbm.at[pl.ds(0, _CH)],
                                  rows_v, gsem).wait()

            @pl.loop(0, _WB)
            def _(j):
                src = rows_v.at[pl.ds(j * _W, _W)]
                pltpu.async_copy(src, s_sh.at[col_v.at[0, j]],
                                 ssem, add=True)
                pltpu.async_copy(ones_v, c_sh.at[col_v.at[0, j]],
                                 csem, add=True)

            pltpu.make_async_copy(rows_v, s_sh.at[pl.ds(0, _CH)],
                                  ssem).wait()
            pltpu.make_async_copy(zc_hbm.at[pl.ds(0, _CH)],
                                  c_sh.at[pl.ds(0, _CH)], csem).wait()

        plsc.subcore_barrier()
        lanes = lax.iota(jnp.int32, 16)

        def drain_chunk(j):
            roff = s * _SST + j * _TQ
            for k in range(16):
                pltpu.make_async_copy(
                    t_v.at[j % 2, k],
                    s_out.at[c, k, pl.ds(roff, _TQ)], gsem).wait()

        @pl.loop(0, 17)
        def _(q):
            @pl.when(q >= 2)
            def _():
                drain_chunk(q - 2)

            roff = s * _SST + q * _TQ
            pltpu.sync_copy(s_sh.at[pl.ds(roff, _TQ)],
                            rows_v.at[pl.ds(0, _TQ)])

            @pl.loop(0, _TQ)
            def _(r):
                plsc.store_scatter(t_v.at[q % 2],
                                   [lanes, jnp.full((16,), r, jnp.int32)],
                                   rows_v[r])

            for k in range(16):
                pltpu.async_copy(t_v.at[q % 2, k],
                                 s_out.at[c, k, pl.ds(roff, _TQ)], gsem)

        drain_chunk(15)
        drain_chunk(16)
        pltpu.sync_copy(c_sh.at[pl.ds(s * _CST, _CST)], c_out.at[c, s])

    return k(glpe, row3d, col3d, zs, zc, ones)


def _tc_combine(x_clique, deg2d, tree_lpe, seg, cnt, demb_pad,
                w1, b1, w2, b2, tw, tb, lw_aug):
    def body(x_ref, d_ref, t_ref, s_ref, c_ref, de_ref, w1_ref, b1_ref,
             w2_ref, b2_ref, tw_ref, tb_ref, lwa_ref, o_ref):
        t3 = jax.nn.relu(
            jnp.dot(de_ref[...], w1_ref[...],
                    preferred_element_type=jnp.float32) + b1_ref[...])
        t4 = jnp.dot(t3, w2_ref[...], preferred_element_type=jnp.float32)
        iota = lax.broadcasted_iota(jnp.int32, (128, _R), 0)
        onehot_t = (d_ref[...] == iota).astype(jnp.float32)
        deg_part = lax.dot_general(onehot_t, t4, (((0,), (0,)), ((), ())),
                                   preferred_element_type=jnp.float32)
        base = (jnp.dot(x_ref[...], w2_ref[...],
                        preferred_element_type=jnp.float32)
                + deg_part + b2_ref[...])
        t = t_ref[...]
        t = jnp.where(t == t, t, 0.0)
        tree_pe = jnp.dot(t, tw_ref[...],
                          preferred_element_type=jnp.float32) + tb_ref[...]
        cn = c_ref[0] + c_ref[1]
        recip = 1.0 / jnp.maximum(cn, 1.0)
        mn = jnp.minimum(cn, 1.0)
        ssum = s_ref[0] + s_ref[1]
        aug = jnp.concatenate([ssum * recip, mn], axis=0)
        pe = lax.dot_general(aug, lwa_ref[...],
                             (((0,), (0,)), ((), ())),
                             preferred_element_type=jnp.float32)
        o_ref[...] = base + jnp.concatenate([pe, tree_pe], axis=1)

    full = lambda shape: pl.BlockSpec(shape, lambda i: (0,) * len(shape))
    return pl.pallas_call(
        body,
        grid=(49,),
        in_specs=[
            pl.BlockSpec((_R, _H), lambda i: (i, 0)),
            pl.BlockSpec((1, _R), lambda i: (0, i)),
            pl.BlockSpec((_R, _PE), lambda i: (i, 0)),
            pl.BlockSpec((2, _PE, _R), lambda i: (0, 0, i)),
            pl.BlockSpec((2, 1, _R), lambda i: (0, 0, i)),
            full((128, _H)),
            full((_H, _H)),
            full((1, _H)),
            full((_H, _H)),
            full((1, _H)),
            full((_PE, _H // 2)),
            full((1, _H // 2)),
            full((_PE + 1, _H // 2)),
        ],
        out_specs=pl.BlockSpec((_R, _H), lambda i: (i, 0)),
        out_shape=jax.ShapeDtypeStruct((_N, _H), jnp.float32),
    )(x_clique, deg2d, tree_lpe, seg, cnt, demb_pad,
      w1, b1, w2, b2, tw, tb, lw_aug)


def kernel(x_clique, tree_degree, tree_lpe, graph_lpe, atom2clique_index,
           deg_emb, deg_lin_w, deg_lin_b, deg_merge_w, deg_merge_b,
           tree_lpe_w, tree_lpe_b, lpe_w, lpe_b):
    padi = jnp.arange(_EP - _E, dtype=jnp.int32)
    padc = _N + padi % (_NS - _N)
    pad = jnp.stack([padi * 997 % _N, padc])
    a2cp = jnp.concatenate([atom2clique_index, pad], axis=1)
    row3d = a2cp[0].reshape(_NBLK, _WB, _W)
    col3d = a2cp[1].reshape(_NBLK, _WB, _W)
    zs = jnp.zeros((_SST, _PE), jnp.float32)
    zc = jnp.zeros((_CST,), jnp.float32)
    ones = jnp.ones((_W,), jnp.float32)
    seg, cnt = _sc_segsum(graph_lpe, row3d, col3d, zs, zc, ones)
    cnt = cnt.reshape(2, 1, _NC)

    deg2d = tree_degree.reshape(1, _N)
    demb_pad = jnp.zeros((128, _H), jnp.float32).at[:100].set(deg_emb)
    lw_aug = jnp.concatenate([lpe_w, lpe_b.reshape(1, _H // 2)], axis=0)
    return _tc_combine(
        x_clique, deg2d, tree_lpe, seg, cnt, demb_pad,
        deg_lin_w, deg_lin_b.reshape(1, _H),
        deg_merge_w, deg_merge_b.reshape(1, _H),
        tree_lpe_w, tree_lpe_b.reshape(1, _H // 2), lw_aug)
